# baseline (speedup 1.0000x reference)
import functools

import jax
import jax.numpy as jnp
from jax import lax
from jax.experimental import pallas as pl
from jax.experimental.pallas import tpu as pltpu
from jax.experimental.pallas import tpu_sc as plsc

N = 100000
D = 128
OUT = 128
S = 512
R = 2000
NB = N // R
GW = 256
NWIN = 4

NC = 2
NS = 16
NW = NC * NS
CH = 128
NFULL = N // CH
TAIL = N - NFULL * CH
ITERS = (NFULL + NW - 1) // NW


def _sc_segsum_body(x_hbm, ids_hbm, zsum_hbm, sums_hbm,
                    xbuf, idxbuf, xtail, idxtail, acc):
    cid = lax.axis_index("c")
    sid = lax.axis_index("s")
    wid = cid * NS + sid

    @pl.when(sid == 0)
    def _init():
        pltpu.sync_copy(zsum_hbm, acc)

    plsc.subcore_barrier()

    def step(it, carry):
        c = wid + it * NW

        @pl.when(c < NFULL)
        def _chunk():
            off = c * CH
            pltpu.sync_copy(x_hbm.at[pl.ds(off, CH)], xbuf)
            pltpu.sync_copy(ids_hbm.at[pl.ds(off, CH)], idxbuf)
            pltpu.sync_copy(xbuf, acc.at[idxbuf], add=True)

        return carry

    lax.fori_loop(0, ITERS, step, 0)

    @pl.when(wid == 0)
    def _tail():
        pltpu.sync_copy(x_hbm.at[pl.ds(NFULL * CH, TAIL)], xtail)
        pltpu.sync_copy(ids_hbm.at[pl.ds(NFULL * CH, TAIL)], idxtail)
        pltpu.sync_copy(xtail, acc.at[idxtail], add=True)

    plsc.subcore_barrier()

    @pl.when(sid == 0)
    def _out():
        pltpu.sync_copy(acc, sums_hbm.at[cid])


def _make_sc_segsum():
    return functools.partial(
        pl.kernel,
        mesh=plsc.VectorSubcoreMesh(core_axis_name="c", subcore_axis_name="s"),
        out_type=jax.ShapeDtypeStruct((NC, S, D), jnp.float32),
        scratch_types=[
            pltpu.VMEM((CH, D), jnp.float32),
            pltpu.VMEM((CH,), jnp.int32),
            pltpu.VMEM((TAIL, D), jnp.float32),
            pltpu.VMEM((TAIL,), jnp.int32),
            pltpu.VMEM_SHARED((S, D), jnp.float32),
        ],
    )(_sc_segsum_body)


_SC_CACHE = {}


def _sc_segsum(*args):
    if "k" not in _SC_CACHE:
        _SC_CACHE["k"] = _make_sc_segsum()
    return _SC_CACHE["k"](*args)


def _counts_body(ids_ref, out_ref, acc_ref):
    i = pl.program_id(0)
    oh = (ids_ref[...] == lax.broadcasted_iota(jnp.int32, (R, S), 1)
          ).astype(jnp.bfloat16)
    part = lax.dot_general(oh, jnp.ones((R, 8), jnp.bfloat16),
                           (((0,), (0,)), ((), ())),
                           preferred_element_type=jnp.float32)

    @pl.when(i == 0)
    def _z():
        acc_ref[...] = jnp.zeros_like(acc_ref)

    acc_ref[...] += part

    @pl.when(i == NB - 1)
    def _w():
        out_ref[...] = acc_ref[...]


def _g_body(sums_ref, cnts_ref, w2t_ref, b_ref, gdup_ref, gfull_ref):
    ssum = sums_ref[0] + sums_ref[1]
    counts = cnts_ref[...][:, 0:1]
    mean = ssum * (1.0 / jnp.maximum(counts, 1.0))
    g = jnp.dot(mean, w2t_ref[...],
                preferred_element_type=jnp.float32) + b_ref[...]
    gb = g.astype(jnp.bfloat16)
    gfull_ref[...] = gb
    gdup_ref[0] = gb[0:256]
    gdup_ref[1] = gb[128:384]
    gdup_ref[2] = gb[256:512]
    gdup_ref[3] = jnp.concatenate(
        [gb[384:512], jnp.zeros((128, OUT), jnp.bfloat16)], axis=0)


def _fused_body(pref_ref, x_ref, ids_ref, w1t_ref, gdup_ref, gfull_ref,
                out_ref):
    i = pl.program_id(0)
    q0 = pref_ref[0, i]
    wide = pref_ref[1, i]
    ids = ids_ref[...]
    main = jnp.dot(x_ref[...], w1t_ref[...],
                   preferred_element_type=jnp.float32)

    @pl.when(wide == 0)
    def _fast():
        rel = ids - q0 * 128
        ohw = (rel == lax.broadcasted_iota(jnp.int32, (R, GW), 1)
               ).astype(jnp.bfloat16)
        out_ref[...] = main + jnp.dot(ohw, gdup_ref[0],
                                      preferred_element_type=jnp.float32)

    @pl.when(wide == 1)
    def _slow():
        oh = (ids == lax.broadcasted_iota(jnp.int32, (R, S), 1)
              ).astype(jnp.bfloat16)
        out_ref[...] = main + jnp.dot(oh, gfull_ref[...],
                                      preferred_element_type=jnp.float32)


@jax.jit
def kernel(x, batch_id, W, b):
    ids = batch_id.astype(jnp.int32)
    ids2d = ids.reshape(N, 1)
    wt = W.T
    w1t = wt[:D]
    w2t = wt[D:]
    brow = b.reshape(1, OUT)
    zsum = jnp.zeros((S, D), jnp.float32)

    sums = _sc_segsum(x, ids, zsum)

    cnts = pl.pallas_call(
        _counts_body,
        grid=(NB,),
        in_specs=[pl.BlockSpec((R, 1), lambda i: (i, 0))],
        out_specs=pl.BlockSpec((S, 8), lambda i: (0, 0)),
        out_shape=jax.ShapeDtypeStruct((S, 8), jnp.float32),
        scratch_shapes=[pltpu.VMEM((S, 8), jnp.float32)],
    )(ids2d)

    gdup, gfull = pl.pallas_call(
        _g_body,
        out_shape=[jax.ShapeDtypeStruct((NWIN, GW, OUT), jnp.bfloat16),
                   jax.ShapeDtypeStruct((S, OUT), jnp.bfloat16)],
    )(sums, cnts, w2t, brow)

    starts = ids[::R]
    ends = ids[R - 1::R]
    q0 = starts // 128
    wide = (ends - q0 * 128 >= GW).astype(jnp.int32)
    pref = jnp.stack([q0, wide])

    grid_spec = pltpu.PrefetchScalarGridSpec(
        num_scalar_prefetch=1,
        grid=(NB,),
        in_specs=[
            pl.BlockSpec((R, D), lambda i, p: (i, 0)),
            pl.BlockSpec((R, 1), lambda i, p: (i, 0)),
            pl.BlockSpec((D, OUT), lambda i, p: (0, 0)),
            pl.BlockSpec((1, GW, OUT), lambda i, p: (p[0, i], 0, 0)),
            pl.BlockSpec((S, OUT), lambda i, p: (0, 0)),
        ],
        out_specs=pl.BlockSpec((R, OUT), lambda i, p: (i, 0)),
    )
    return pl.pallas_call(
        _fused_body,
        grid_spec=grid_spec,
        out_shape=jax.ShapeDtypeStruct((N, OUT), jnp.float32),
    )(pref, x, ids2d, w1t, gdup, gfull)


# speedup vs baseline: 1.3587x; 1.3587x over previous
import functools

import jax
import jax.numpy as jnp
from jax import lax
from jax.experimental import pallas as pl
from jax.experimental.pallas import tpu as pltpu
from jax.experimental.pallas import tpu_sc as plsc

N = 100000
D = 128
OUT = 128
S = 512
R = 2000
NB = N // R
GW = 256
NWIN = 4

NC = 2
NS = 16
NW = NC * NS
CH = 128
NFULL = N // CH
TAIL = N - NFULL * CH
ITERS = (NFULL + NW - 1) // NW


def _sc_segsum_body(x_hbm, ids_hbm, zsum_hbm, sums_hbm,
                    xbuf, idxbuf, xtail, idxtail, acc):
    cid = lax.axis_index("c")
    sid = lax.axis_index("s")
    wid = cid * NS + sid

    @pl.when(sid == 0)
    def _init():
        pltpu.sync_copy(zsum_hbm, acc)

    plsc.subcore_barrier()

    def step(it, carry):
        c = wid + it * NW

        @pl.when(c < NFULL)
        def _chunk():
            off = c * CH
            pltpu.sync_copy(x_hbm.at[pl.ds(off, CH)], xbuf)
            pltpu.sync_copy(ids_hbm.at[pl.ds(off, CH)], idxbuf)
            pltpu.sync_copy(xbuf, acc.at[idxbuf], add=True)

        return carry

    lax.fori_loop(0, ITERS, step, 0)

    @pl.when(wid == 0)
    def _tail():
        pltpu.sync_copy(x_hbm.at[pl.ds(NFULL * CH, TAIL)], xtail)
        pltpu.sync_copy(ids_hbm.at[pl.ds(NFULL * CH, TAIL)], idxtail)
        pltpu.sync_copy(xtail, acc.at[idxtail], add=True)

    plsc.subcore_barrier()

    @pl.when(sid == 0)
    def _out():
        pltpu.sync_copy(acc, sums_hbm.at[cid])


def _make_sc_segsum():
    return functools.partial(
        pl.kernel,
        mesh=plsc.VectorSubcoreMesh(core_axis_name="c", subcore_axis_name="s"),
        out_type=jax.ShapeDtypeStruct((NC, S, D), jnp.float32),
        scratch_types=[
            pltpu.VMEM((CH, D), jnp.float32),
            pltpu.VMEM((CH,), jnp.int32),
            pltpu.VMEM((TAIL, D), jnp.float32),
            pltpu.VMEM((TAIL,), jnp.int32),
            pltpu.VMEM_SHARED((S, D), jnp.float32),
        ],
    )(_sc_segsum_body)


_SC_CACHE = {}


def _sc_segsum(*args):
    if "k" not in _SC_CACHE:
        _SC_CACHE["k"] = _make_sc_segsum()
    return _SC_CACHE["k"](*args)


def _counts_body(ids_ref, out_ref, acc_ref):
    i = pl.program_id(0)
    ids_row = ids_ref[0]
    ohT = (ids_row == lax.broadcasted_iota(jnp.int32, (S, R), 0)
           ).astype(jnp.bfloat16)
    part = jnp.dot(ohT, jnp.ones((R, 8), jnp.bfloat16),
                   preferred_element_type=jnp.float32)

    @pl.when(i == 0)
    def _z():
        acc_ref[...] = jnp.zeros_like(acc_ref)

    acc_ref[...] += part

    @pl.when(i == NB - 1)
    def _w():
        out_ref[...] = acc_ref[...]


def _g_body(sums_ref, cnts_ref, w2t_ref, b_ref, gdup_ref, gfull_ref):
    ssum = sums_ref[0] + sums_ref[1]
    counts = cnts_ref[...][:, 0:1]
    mean = ssum * (1.0 / jnp.maximum(counts, 1.0))
    g = jnp.dot(mean, w2t_ref[...],
                preferred_element_type=jnp.float32) + b_ref[...]
    gb = g.astype(jnp.bfloat16)
    gfull_ref[...] = gb
    gdup_ref[0] = gb[0:256]
    gdup_ref[1] = gb[128:384]
    gdup_ref[2] = gb[256:512]
    gdup_ref[3] = jnp.concatenate(
        [gb[384:512], jnp.zeros((128, OUT), jnp.bfloat16)], axis=0)


def _fused_body(pref_ref, x_ref, ids_ref, w1t_ref, gdup_ref, gfull_ref,
                out_ref):
    i = pl.program_id(0)
    q0 = pref_ref[0, i]
    wide = pref_ref[1, i]
    ids_row = ids_ref[0]
    main = jnp.dot(x_ref[...], w1t_ref[...],
                   preferred_element_type=jnp.float32)

    @pl.when(wide == 0)
    def _fast():
        rel = ids_row - q0 * 128
        ohT = (rel == lax.broadcasted_iota(jnp.int32, (GW, R), 0)
               ).astype(jnp.bfloat16)
        gath = lax.dot_general(ohT, gdup_ref[0], (((0,), (0,)), ((), ())),
                               preferred_element_type=jnp.float32)
        out_ref[...] = main + gath

    @pl.when(wide == 1)
    def _slow():
        ohT = (ids_row == lax.broadcasted_iota(jnp.int32, (S, R), 0)
               ).astype(jnp.bfloat16)
        gath = lax.dot_general(ohT, gfull_ref[...], (((0,), (0,)), ((), ())),
                               preferred_element_type=jnp.float32)
        out_ref[...] = main + gath


@jax.jit
def kernel(x, batch_id, W, b):
    ids = batch_id.astype(jnp.int32)
    ids3 = ids.reshape(NB, 1, R)
    wt = W.T
    w1t = wt[:D]
    w2t = wt[D:]
    brow = b.reshape(1, OUT)
    zsum = jnp.zeros((S, D), jnp.float32)

    sums = _sc_segsum(x, ids, zsum)

    cnts = pl.pallas_call(
        _counts_body,
        grid=(NB,),
        in_specs=[pl.BlockSpec((1, 1, R), lambda i: (i, 0, 0))],
        out_specs=pl.BlockSpec((S, 8), lambda i: (0, 0)),
        out_shape=jax.ShapeDtypeStruct((S, 8), jnp.float32),
        scratch_shapes=[pltpu.VMEM((S, 8), jnp.float32)],
    )(ids3)

    gdup, gfull = pl.pallas_call(
        _g_body,
        out_shape=[jax.ShapeDtypeStruct((NWIN, GW, OUT), jnp.bfloat16),
                   jax.ShapeDtypeStruct((S, OUT), jnp.bfloat16)],
    )(sums, cnts, w2t, brow)

    starts = ids[::R]
    ends = ids[R - 1::R]
    q0 = starts // 128
    wide = (ends - q0 * 128 >= GW).astype(jnp.int32)
    pref = jnp.stack([q0, wide])

    grid_spec = pltpu.PrefetchScalarGridSpec(
        num_scalar_prefetch=1,
        grid=(NB,),
        in_specs=[
            pl.BlockSpec((R, D), lambda i, p: (i, 0)),
            pl.BlockSpec((1, 1, R), lambda i, p: (i, 0, 0)),
            pl.BlockSpec((D, OUT), lambda i, p: (0, 0)),
            pl.BlockSpec((1, GW, OUT), lambda i, p: (p[0, i], 0, 0)),
            pl.BlockSpec((S, OUT), lambda i, p: (0, 0)),
        ],
        out_specs=pl.BlockSpec((R, OUT), lambda i, p: (i, 0)),
    )
    return pl.pallas_call(
        _fused_body,
        grid_spec=grid_spec,
        out_shape=jax.ShapeDtypeStruct((N, OUT), jnp.float32),
    )(pref, x, ids3, w1t, gdup, gfull)


# speedup vs baseline: 1.5925x; 1.1721x over previous
import functools

import jax
import jax.numpy as jnp
from jax import lax
from jax.experimental import pallas as pl
from jax.experimental.pallas import tpu as pltpu
from jax.experimental.pallas import tpu_sc as plsc

N = 100000
D = 128
OUT = 128
S = 512
R = 2000
NB = N // R
GW = 256
NWIN = 4

NC = 2
NS = 16
NW = NC * NS
CH = 128
NFULL = N // CH
TAIL = N - NFULL * CH
ITERS = (NFULL + NW - 1) // NW


def _sc_segsum_body(x_hbm, ids_hbm, zsum_hbm, sums_hbm,
                    xbuf, idxbuf, xtail, idxtail, acc,
                    semx0, semx1, semi0, semi1):
    cid = lax.axis_index("c")
    sid = lax.axis_index("s")
    wid = cid * NS + sid
    semx = [semx0, semx1]
    semi = [semi0, semi1]

    @pl.when(sid == 0)
    def _init():
        pltpu.sync_copy(zsum_hbm, acc)

    def _start(slot, c):
        off = c * CH
        pltpu.async_copy(x_hbm.at[pl.ds(off, CH)], xbuf.at[slot], semx[slot])
        pltpu.async_copy(ids_hbm.at[pl.ds(off, CH)], idxbuf.at[slot],
                         semi[slot])

    def _wait(slot, c):
        off = c * CH
        pltpu.make_async_copy(x_hbm.at[pl.ds(off, CH)], xbuf.at[slot],
                              semx[slot]).wait()
        pltpu.make_async_copy(ids_hbm.at[pl.ds(off, CH)], idxbuf.at[slot],
                              semi[slot]).wait()

    for b in range(2):
        cpr = wid + b * NW

        @pl.when(cpr < NFULL)
        def _p(cpr=cpr, b=b):
            _start(b, cpr)

    plsc.subcore_barrier()

    def step(k, carry):
        for b in range(2):
            it = 2 * k + b
            c = wid + it * NW

            @pl.when(c < NFULL)
            def _chunk(c=c, b=b):
                _wait(b, c)
                pltpu.sync_copy(xbuf.at[b], acc.at[idxbuf.at[b]], add=True)
                c2 = c + 2 * NW

                @pl.when(c2 < NFULL)
                def _n(c2=c2, b=b):
                    _start(b, c2)

        return carry

    lax.fori_loop(0, (ITERS + 1) // 2, step, 0)

    @pl.when(wid == 0)
    def _tail():
        pltpu.sync_copy(x_hbm.at[pl.ds(NFULL * CH, TAIL)], xtail)
        pltpu.sync_copy(ids_hbm.at[pl.ds(NFULL * CH, TAIL)], idxtail)
        pltpu.sync_copy(xtail, acc.at[idxtail], add=True)

    plsc.subcore_barrier()

    @pl.when(sid == 0)
    def _out():
        pltpu.sync_copy(acc, sums_hbm.at[cid])


def _make_sc_segsum():
    return functools.partial(
        pl.kernel,
        mesh=plsc.VectorSubcoreMesh(core_axis_name="c", subcore_axis_name="s"),
        out_type=jax.ShapeDtypeStruct((NC, S, D), jnp.float32),
        scratch_types=[
            pltpu.VMEM((2, CH, D), jnp.float32),
            pltpu.VMEM((2, CH), jnp.int32),
            pltpu.VMEM((TAIL, D), jnp.float32),
            pltpu.VMEM((TAIL,), jnp.int32),
            pltpu.VMEM_SHARED((S, D), jnp.float32),
            pltpu.SemaphoreType.DMA,
            pltpu.SemaphoreType.DMA,
            pltpu.SemaphoreType.DMA,
            pltpu.SemaphoreType.DMA,
        ],
    )(_sc_segsum_body)


_SC_CACHE = {}


def _sc_segsum(*args):
    if "k" not in _SC_CACHE:
        _SC_CACHE["k"] = _make_sc_segsum()
    return _SC_CACHE["k"](*args)


def _counts_body(ids_ref, out_ref, acc_ref):
    i = pl.program_id(0)
    ids_row = ids_ref[0]
    ohT = (ids_row == lax.broadcasted_iota(jnp.int32, (S, R), 0)
           ).astype(jnp.bfloat16)
    part = jnp.dot(ohT, jnp.ones((R, 8), jnp.bfloat16),
                   preferred_element_type=jnp.float32)

    @pl.when(i == 0)
    def _z():
        acc_ref[...] = jnp.zeros_like(acc_ref)

    acc_ref[...] += part

    @pl.when(i == NB - 1)
    def _w():
        out_ref[...] = acc_ref[...]


def _g_body(sums_ref, cnts_ref, w2t_ref, b_ref, gdup_ref, gfull_ref):
    ssum = sums_ref[0] + sums_ref[1]
    counts = cnts_ref[...][:, 0:1]
    mean = ssum * (1.0 / jnp.maximum(counts, 1.0))
    g = jnp.dot(mean, w2t_ref[...],
                preferred_element_type=jnp.float32) + b_ref[...]
    gb = g.astype(jnp.bfloat16)
    gfull_ref[...] = gb
    gdup_ref[0] = gb[0:256]
    gdup_ref[1] = gb[128:384]
    gdup_ref[2] = gb[256:512]
    gdup_ref[3] = jnp.concatenate(
        [gb[384:512], jnp.zeros((128, OUT), jnp.bfloat16)], axis=0)


def _fused_body(pref_ref, x_ref, ids_ref, w1t_ref, gdup_ref, gfull_ref,
                out_ref):
    i = pl.program_id(0)
    q0 = pref_ref[0, i]
    wide = pref_ref[1, i]
    ids_row = ids_ref[0]
    main = jnp.dot(x_ref[...], w1t_ref[...],
                   preferred_element_type=jnp.float32)

    @pl.when(wide == 0)
    def _fast():
        rel = ids_row - q0 * 128
        ohT = (rel == lax.broadcasted_iota(jnp.int32, (GW, R), 0)
               ).astype(jnp.bfloat16)
        gath = lax.dot_general(ohT, gdup_ref[0], (((0,), (0,)), ((), ())),
                               preferred_element_type=jnp.float32)
        out_ref[...] = main + gath

    @pl.when(wide == 1)
    def _slow():
        ohT = (ids_row == lax.broadcasted_iota(jnp.int32, (S, R), 0)
               ).astype(jnp.bfloat16)
        gath = lax.dot_general(ohT, gfull_ref[...], (((0,), (0,)), ((), ())),
                               preferred_element_type=jnp.float32)
        out_ref[...] = main + gath


@jax.jit
def kernel(x, batch_id, W, b):
    ids = batch_id.astype(jnp.int32)
    ids3 = ids.reshape(NB, 1, R)
    wt = W.T
    w1t = wt[:D]
    w2t = wt[D:]
    brow = b.reshape(1, OUT)
    zsum = jnp.zeros((S, D), jnp.float32)

    sums = _sc_segsum(x, ids, zsum)

    cnts = pl.pallas_call(
        _counts_body,
        grid=(NB,),
        in_specs=[pl.BlockSpec((1, 1, R), lambda i: (i, 0, 0))],
        out_specs=pl.BlockSpec((S, 8), lambda i: (0, 0)),
        out_shape=jax.ShapeDtypeStruct((S, 8), jnp.float32),
        scratch_shapes=[pltpu.VMEM((S, 8), jnp.float32)],
    )(ids3)

    gdup, gfull = pl.pallas_call(
        _g_body,
        out_shape=[jax.ShapeDtypeStruct((NWIN, GW, OUT), jnp.bfloat16),
                   jax.ShapeDtypeStruct((S, OUT), jnp.bfloat16)],
    )(sums, cnts, w2t, brow)

    starts = ids[::R]
    ends = ids[R - 1::R]
    q0 = starts // 128
    wide = (ends - q0 * 128 >= GW).astype(jnp.int32)
    pref = jnp.stack([q0, wide])

    grid_spec = pltpu.PrefetchScalarGridSpec(
        num_scalar_prefetch=1,
        grid=(NB,),
        in_specs=[
            pl.BlockSpec((R, D), lambda i, p: (i, 0)),
            pl.BlockSpec((1, 1, R), lambda i, p: (i, 0, 0)),
            pl.BlockSpec((D, OUT), lambda i, p: (0, 0)),
            pl.BlockSpec((1, GW, OUT), lambda i, p: (p[0, i], 0, 0)),
            pl.BlockSpec((S, OUT), lambda i, p: (0, 0)),
        ],
        out_specs=pl.BlockSpec((R, OUT), lambda i, p: (i, 0)),
    )
    return pl.pallas_call(
        _fused_body,
        grid_spec=grid_spec,
        out_shape=jax.ShapeDtypeStruct((N, OUT), jnp.float32),
    )(pref, x, ids3, w1t, gdup, gfull)


# speedup vs baseline: 1.9686x; 1.2362x over previous
import functools

import jax
import jax.numpy as jnp
from jax import lax
from jax.experimental import pallas as pl
from jax.experimental.pallas import tpu as pltpu
from jax.experimental.pallas import tpu_sc as plsc

N = 100000
D = 128
OUT = 128
S = 512
R = 4000
NB = N // R
GW = 256
NWIN = 4

NC = 2
NS = 16
NW = NC * NS
CH = 128
NFULL = N // CH
TAIL = N - NFULL * CH
ITERS = (NFULL + NW - 1) // NW


def _sc_segsum_body(x_hbm, ids_hbm, zsum_hbm, sums_hbm,
                    xbuf, idxbuf, xtail, idxtail, acc,
                    semx0, semx1, semi0, semi1):
    cid = lax.axis_index("c")
    sid = lax.axis_index("s")
    wid = cid * NS + sid
    semx = [semx0, semx1]
    semi = [semi0, semi1]

    @pl.when(sid == 0)
    def _init():
        pltpu.sync_copy(zsum_hbm, acc)

    def _start(slot, c):
        off = c * CH
        pltpu.async_copy(x_hbm.at[pl.ds(off, CH)], xbuf.at[slot], semx[slot])
        pltpu.async_copy(ids_hbm.at[pl.ds(off, CH)], idxbuf.at[slot],
                         semi[slot])

    def _wait(slot, c):
        off = c * CH
        pltpu.make_async_copy(x_hbm.at[pl.ds(off, CH)], xbuf.at[slot],
                              semx[slot]).wait()
        pltpu.make_async_copy(ids_hbm.at[pl.ds(off, CH)], idxbuf.at[slot],
                              semi[slot]).wait()

    for b in range(2):
        cpr = wid + b * NW

        @pl.when(cpr < NFULL)
        def _p(cpr=cpr, b=b):
            _start(b, cpr)

    plsc.subcore_barrier()

    def step(k, carry):
        for b in range(2):
            it = 2 * k + b
            c = wid + it * NW

            @pl.when(c < NFULL)
            def _chunk(c=c, b=b):
                _wait(b, c)
                pltpu.sync_copy(xbuf.at[b], acc.at[idxbuf.at[b]], add=True)
                c2 = c + 2 * NW

                @pl.when(c2 < NFULL)
                def _n(c2=c2, b=b):
                    _start(b, c2)

        return carry

    lax.fori_loop(0, (ITERS + 1) // 2, step, 0)

    @pl.when(wid == 0)
    def _tail():
        pltpu.sync_copy(x_hbm.at[pl.ds(NFULL * CH, TAIL)], xtail)
        pltpu.sync_copy(ids_hbm.at[pl.ds(NFULL * CH, TAIL)], idxtail)
        pltpu.sync_copy(xtail, acc.at[idxtail], add=True)

    plsc.subcore_barrier()

    @pl.when(sid == 0)
    def _out():
        pltpu.sync_copy(acc, sums_hbm.at[cid])


def _make_sc_segsum():
    return functools.partial(
        pl.kernel,
        mesh=plsc.VectorSubcoreMesh(core_axis_name="c", subcore_axis_name="s"),
        out_type=jax.ShapeDtypeStruct((NC, S, D), jnp.float32),
        scratch_types=[
            pltpu.VMEM((2, CH, D), jnp.float32),
            pltpu.VMEM((2, CH), jnp.int32),
            pltpu.VMEM((TAIL, D), jnp.float32),
            pltpu.VMEM((TAIL,), jnp.int32),
            pltpu.VMEM_SHARED((S, D), jnp.float32),
            pltpu.SemaphoreType.DMA,
            pltpu.SemaphoreType.DMA,
            pltpu.SemaphoreType.DMA,
            pltpu.SemaphoreType.DMA,
        ],
    )(_sc_segsum_body)


_SC_CACHE = {}


def _sc_segsum(*args):
    if "k" not in _SC_CACHE:
        _SC_CACHE["k"] = _make_sc_segsum()
    return _SC_CACHE["k"](*args)


def _counts_body(ids_ref, out_ref, acc_ref):
    i = pl.program_id(0)
    ids_row = ids_ref[0]
    ohT = (ids_row == lax.broadcasted_iota(jnp.int32, (S, R), 0)
           ).astype(jnp.bfloat16)
    part = jnp.dot(ohT, jnp.ones((R, 8), jnp.bfloat16),
                   preferred_element_type=jnp.float32)

    @pl.when(i == 0)
    def _z():
        acc_ref[...] = jnp.zeros_like(acc_ref)

    acc_ref[...] += part

    @pl.when(i == NB - 1)
    def _w():
        out_ref[...] = acc_ref[...]


def _g_body(sums_ref, cnts_ref, w2t_ref, b_ref, gdup_ref, gfull_ref):
    ssum = sums_ref[0] + sums_ref[1]
    counts = cnts_ref[...][:, 0:1]
    mean = ssum * (1.0 / jnp.maximum(counts, 1.0))
    g = jnp.dot(mean, w2t_ref[...],
                preferred_element_type=jnp.float32) + b_ref[...]
    gb = g.astype(jnp.bfloat16)
    gfull_ref[...] = gb
    gdup_ref[0] = gb[0:256]
    gdup_ref[1] = gb[128:384]
    gdup_ref[2] = gb[256:512]
    gdup_ref[3] = jnp.concatenate(
        [gb[384:512], jnp.zeros((128, OUT), jnp.bfloat16)], axis=0)


def _fused_body(pref_ref, x_ref, ids_ref, w1t_ref, gdup_ref, gfull_ref,
                out_ref):
    i = pl.program_id(0)
    q0 = pref_ref[0, i]
    wide = pref_ref[1, i]
    ids_row = ids_ref[0]
    main = jnp.dot(x_ref[...], w1t_ref[...],
                   preferred_element_type=jnp.float32)

    @pl.when(wide == 0)
    def _fast():
        rel = ids_row - q0 * 128
        ohT = (rel == lax.broadcasted_iota(jnp.int32, (GW, R), 0)
               ).astype(jnp.bfloat16)
        gath = lax.dot_general(ohT, gdup_ref[0], (((0,), (0,)), ((), ())),
                               preferred_element_type=jnp.float32)
        out_ref[...] = main + gath

    @pl.when(wide == 1)
    def _slow():
        ohT = (ids_row == lax.broadcasted_iota(jnp.int32, (S, R), 0)
               ).astype(jnp.bfloat16)
        gath = lax.dot_general(ohT, gfull_ref[...], (((0,), (0,)), ((), ())),
                               preferred_element_type=jnp.float32)
        out_ref[...] = main + gath


@jax.jit
def kernel(x, batch_id, W, b):
    ids = batch_id.astype(jnp.int32)
    ids3 = ids.reshape(NB, 1, R)
    wt = W.T
    w1t = wt[:D]
    w2t = wt[D:]
    brow = b.reshape(1, OUT)
    zsum = jnp.zeros((S, D), jnp.float32)

    sums = _sc_segsum(x, ids, zsum)

    cnts = pl.pallas_call(
        _counts_body,
        grid=(NB,),
        in_specs=[pl.BlockSpec((1, 1, R), lambda i: (i, 0, 0))],
        out_specs=pl.BlockSpec((S, 8), lambda i: (0, 0)),
        out_shape=jax.ShapeDtypeStruct((S, 8), jnp.float32),
        scratch_shapes=[pltpu.VMEM((S, 8), jnp.float32)],
    )(ids3)

    gdup, gfull = pl.pallas_call(
        _g_body,
        out_shape=[jax.ShapeDtypeStruct((NWIN, GW, OUT), jnp.bfloat16),
                   jax.ShapeDtypeStruct((S, OUT), jnp.bfloat16)],
    )(sums, cnts, w2t, brow)

    starts = ids[::R]
    ends = ids[R - 1::R]
    q0 = starts // 128
    wide = (ends - q0 * 128 >= GW).astype(jnp.int32)
    pref = jnp.stack([q0, wide])

    grid_spec = pltpu.PrefetchScalarGridSpec(
        num_scalar_prefetch=1,
        grid=(NB,),
        in_specs=[
            pl.BlockSpec((R, D), lambda i, p: (i, 0)),
            pl.BlockSpec((1, 1, R), lambda i, p: (i, 0, 0)),
            pl.BlockSpec((D, OUT), lambda i, p: (0, 0)),
            pl.BlockSpec((1, GW, OUT), lambda i, p: (p[0, i], 0, 0)),
            pl.BlockSpec((S, OUT), lambda i, p: (0, 0)),
        ],
        out_specs=pl.BlockSpec((R, OUT), lambda i, p: (i, 0)),
    )
    return pl.pallas_call(
        _fused_body,
        grid_spec=grid_spec,
        out_shape=jax.ShapeDtypeStruct((N, OUT), jnp.float32),
    )(pref, x, ids3, w1t, gdup, gfull)


# speedup vs baseline: 2.2093x; 1.1223x over previous
import functools

import jax
import jax.numpy as jnp
from jax import lax
from jax.experimental import pallas as pl
from jax.experimental.pallas import tpu as pltpu
from jax.experimental.pallas import tpu_sc as plsc

N = 100000
D = 128
OUT = 128
S = 512
R = 10000
NB = N // R
GW = 256
NWIN = 4

NC = 2
NS = 16
NW = NC * NS
CH = 128
NFULL = N // CH
TAIL = N - NFULL * CH
ITERS = (NFULL + NW - 1) // NW


def _sc_segsum_body(x_hbm, ids_hbm, zsum_hbm, sums_hbm,
                    xbuf, idxbuf, xtail, idxtail, acc,
                    semx0, semx1, semi0, semi1):
    cid = lax.axis_index("c")
    sid = lax.axis_index("s")
    wid = cid * NS + sid
    semx = [semx0, semx1]
    semi = [semi0, semi1]

    @pl.when(sid == 0)
    def _init():
        pltpu.sync_copy(zsum_hbm, acc)

    def _start(slot, c):
        off = c * CH
        pltpu.async_copy(x_hbm.at[pl.ds(off, CH)], xbuf.at[slot], semx[slot])
        pltpu.async_copy(ids_hbm.at[pl.ds(off, CH)], idxbuf.at[slot],
                         semi[slot])

    def _wait(slot, c):
        off = c * CH
        pltpu.make_async_copy(x_hbm.at[pl.ds(off, CH)], xbuf.at[slot],
                              semx[slot]).wait()
        pltpu.make_async_copy(ids_hbm.at[pl.ds(off, CH)], idxbuf.at[slot],
                              semi[slot]).wait()

    for b in range(2):
        cpr = wid + b * NW

        @pl.when(cpr < NFULL)
        def _p(cpr=cpr, b=b):
            _start(b, cpr)

    plsc.subcore_barrier()

    def step(k, carry):
        for b in range(2):
            it = 2 * k + b
            c = wid + it * NW

            @pl.when(c < NFULL)
            def _chunk(c=c, b=b):
                _wait(b, c)
                pltpu.sync_copy(xbuf.at[b], acc.at[idxbuf.at[b]], add=True)
                c2 = c + 2 * NW

                @pl.when(c2 < NFULL)
                def _n(c2=c2, b=b):
                    _start(b, c2)

        return carry

    lax.fori_loop(0, (ITERS + 1) // 2, step, 0)

    @pl.when(wid == 0)
    def _tail():
        pltpu.sync_copy(x_hbm.at[pl.ds(NFULL * CH, TAIL)], xtail)
        pltpu.sync_copy(ids_hbm.at[pl.ds(NFULL * CH, TAIL)], idxtail)
        pltpu.sync_copy(xtail, acc.at[idxtail], add=True)

    plsc.subcore_barrier()

    @pl.when(sid == 0)
    def _out():
        pltpu.sync_copy(acc, sums_hbm.at[cid])


def _make_sc_segsum():
    return functools.partial(
        pl.kernel,
        mesh=plsc.VectorSubcoreMesh(core_axis_name="c", subcore_axis_name="s"),
        out_type=jax.ShapeDtypeStruct((NC, S, D), jnp.float32),
        scratch_types=[
            pltpu.VMEM((2, CH, D), jnp.float32),
            pltpu.VMEM((2, CH), jnp.int32),
            pltpu.VMEM((TAIL, D), jnp.float32),
            pltpu.VMEM((TAIL,), jnp.int32),
            pltpu.VMEM_SHARED((S, D), jnp.float32),
            pltpu.SemaphoreType.DMA,
            pltpu.SemaphoreType.DMA,
            pltpu.SemaphoreType.DMA,
            pltpu.SemaphoreType.DMA,
        ],
    )(_sc_segsum_body)


_SC_CACHE = {}


def _sc_segsum(*args):
    if "k" not in _SC_CACHE:
        _SC_CACHE["k"] = _make_sc_segsum()
    return _SC_CACHE["k"](*args)


def _counts_body(ids_ref, out_ref, acc_ref):
    i = pl.program_id(0)
    ids_row = ids_ref[0]
    ohT = (ids_row == lax.broadcasted_iota(jnp.int32, (S, R), 0)
           ).astype(jnp.bfloat16)
    part = jnp.dot(ohT, jnp.ones((R, 8), jnp.bfloat16),
                   preferred_element_type=jnp.float32)

    @pl.when(i == 0)
    def _z():
        acc_ref[...] = jnp.zeros_like(acc_ref)

    acc_ref[...] += part

    @pl.when(i == NB - 1)
    def _w():
        out_ref[...] = acc_ref[...]


def _g_body(sums_ref, cnts_ref, w2t_ref, b_ref, gdup_ref, gfull_ref):
    ssum = sums_ref[0] + sums_ref[1]
    counts = cnts_ref[...][:, 0:1]
    mean = ssum * (1.0 / jnp.maximum(counts, 1.0))
    g = jnp.dot(mean, w2t_ref[...],
                preferred_element_type=jnp.float32) + b_ref[...]
    gb = g.astype(jnp.bfloat16)
    gfull_ref[...] = gb
    gdup_ref[0] = gb[0:256]
    gdup_ref[1] = gb[128:384]
    gdup_ref[2] = gb[256:512]
    gdup_ref[3] = jnp.concatenate(
        [gb[384:512], jnp.zeros((128, OUT), jnp.bfloat16)], axis=0)


def _fused_body(pref_ref, x_ref, ids_ref, w1t_ref, gdup_ref, gfull_ref,
                out_ref):
    i = pl.program_id(0)
    q0 = pref_ref[0, i]
    wide = pref_ref[1, i]
    ids_row = ids_ref[0]
    main = jnp.dot(x_ref[...], w1t_ref[...],
                   preferred_element_type=jnp.float32)

    @pl.when(wide == 0)
    def _fast():
        rel = ids_row - q0 * 128
        ohT = (rel == lax.broadcasted_iota(jnp.int32, (GW, R), 0)
               ).astype(jnp.bfloat16)
        gath = lax.dot_general(ohT, gdup_ref[0], (((0,), (0,)), ((), ())),
                               preferred_element_type=jnp.float32)
        out_ref[...] = main + gath

    @pl.when(wide == 1)
    def _slow():
        ohT = (ids_row == lax.broadcasted_iota(jnp.int32, (S, R), 0)
               ).astype(jnp.bfloat16)
        gath = lax.dot_general(ohT, gfull_ref[...], (((0,), (0,)), ((), ())),
                               preferred_element_type=jnp.float32)
        out_ref[...] = main + gath


@jax.jit
def kernel(x, batch_id, W, b):
    ids = batch_id.astype(jnp.int32)
    ids3 = ids.reshape(NB, 1, R)
    wt = W.T
    w1t = wt[:D]
    w2t = wt[D:]
    brow = b.reshape(1, OUT)
    zsum = jnp.zeros((S, D), jnp.float32)

    sums = _sc_segsum(x, ids, zsum)

    cnts = pl.pallas_call(
        _counts_body,
        grid=(NB,),
        in_specs=[pl.BlockSpec((1, 1, R), lambda i: (i, 0, 0))],
        out_specs=pl.BlockSpec((S, 8), lambda i: (0, 0)),
        out_shape=jax.ShapeDtypeStruct((S, 8), jnp.float32),
        scratch_shapes=[pltpu.VMEM((S, 8), jnp.float32)],
    )(ids3)

    gdup, gfull = pl.pallas_call(
        _g_body,
        out_shape=[jax.ShapeDtypeStruct((NWIN, GW, OUT), jnp.bfloat16),
                   jax.ShapeDtypeStruct((S, OUT), jnp.bfloat16)],
    )(sums, cnts, w2t, brow)

    starts = ids[::R]
    ends = ids[R - 1::R]
    q0 = starts // 128
    wide = (ends - q0 * 128 >= GW).astype(jnp.int32)
    pref = jnp.stack([q0, wide])

    grid_spec = pltpu.PrefetchScalarGridSpec(
        num_scalar_prefetch=1,
        grid=(NB,),
        in_specs=[
            pl.BlockSpec((R, D), lambda i, p: (i, 0)),
            pl.BlockSpec((1, 1, R), lambda i, p: (i, 0, 0)),
            pl.BlockSpec((D, OUT), lambda i, p: (0, 0)),
            pl.BlockSpec((1, GW, OUT), lambda i, p: (p[0, i], 0, 0)),
            pl.BlockSpec((S, OUT), lambda i, p: (0, 0)),
        ],
        out_specs=pl.BlockSpec((R, OUT), lambda i, p: (i, 0)),
    )
    return pl.pallas_call(
        _fused_body,
        grid_spec=grid_spec,
        out_shape=jax.ShapeDtypeStruct((N, OUT), jnp.float32),
    )(pref, x, ids3, w1t, gdup, gfull)


# speedup vs baseline: 2.2164x; 1.0032x over previous
import functools

import jax
import jax.numpy as jnp
from jax import lax
from jax.experimental import pallas as pl
from jax.experimental.pallas import tpu as pltpu
from jax.experimental.pallas import tpu_sc as plsc

N = 100000
D = 128
OUT = 128
S = 512
R = 20000
NB = N // R
GW = 256
NWIN = 4

NC = 2
NS = 16
NW = NC * NS
CH = 128
NFULL = N // CH
TAIL = N - NFULL * CH
ITERS = (NFULL + NW - 1) // NW


def _sc_segsum_body(x_hbm, ids_hbm, zsum_hbm, sums_hbm,
                    xbuf, idxbuf, xtail, idxtail, acc,
                    semx0, semx1, semi0, semi1):
    cid = lax.axis_index("c")
    sid = lax.axis_index("s")
    wid = cid * NS + sid
    semx = [semx0, semx1]
    semi = [semi0, semi1]

    @pl.when(sid == 0)
    def _init():
        pltpu.sync_copy(zsum_hbm, acc)

    def _start(slot, c):
        off = c * CH
        pltpu.async_copy(x_hbm.at[pl.ds(off, CH)], xbuf.at[slot], semx[slot])
        pltpu.async_copy(ids_hbm.at[pl.ds(off, CH)], idxbuf.at[slot],
                         semi[slot])

    def _wait(slot, c):
        off = c * CH
        pltpu.make_async_copy(x_hbm.at[pl.ds(off, CH)], xbuf.at[slot],
                              semx[slot]).wait()
        pltpu.make_async_copy(ids_hbm.at[pl.ds(off, CH)], idxbuf.at[slot],
                              semi[slot]).wait()

    for b in range(2):
        cpr = wid + b * NW

        @pl.when(cpr < NFULL)
        def _p(cpr=cpr, b=b):
            _start(b, cpr)

    plsc.subcore_barrier()

    def step(k, carry):
        for b in range(2):
            it = 2 * k + b
            c = wid + it * NW

            @pl.when(c < NFULL)
            def _chunk(c=c, b=b):
                _wait(b, c)
                pltpu.sync_copy(xbuf.at[b], acc.at[idxbuf.at[b]], add=True)
                c2 = c + 2 * NW

                @pl.when(c2 < NFULL)
                def _n(c2=c2, b=b):
                    _start(b, c2)

        return carry

    lax.fori_loop(0, (ITERS + 1) // 2, step, 0)

    @pl.when(wid == 0)
    def _tail():
        pltpu.sync_copy(x_hbm.at[pl.ds(NFULL * CH, TAIL)], xtail)
        pltpu.sync_copy(ids_hbm.at[pl.ds(NFULL * CH, TAIL)], idxtail)
        pltpu.sync_copy(xtail, acc.at[idxtail], add=True)

    plsc.subcore_barrier()

    @pl.when(sid == 0)
    def _out():
        pltpu.sync_copy(acc, sums_hbm.at[cid])


def _make_sc_segsum():
    return functools.partial(
        pl.kernel,
        mesh=plsc.VectorSubcoreMesh(core_axis_name="c", subcore_axis_name="s"),
        out_type=jax.ShapeDtypeStruct((NC, S, D), jnp.float32),
        scratch_types=[
            pltpu.VMEM((2, CH, D), jnp.float32),
            pltpu.VMEM((2, CH), jnp.int32),
            pltpu.VMEM((TAIL, D), jnp.float32),
            pltpu.VMEM((TAIL,), jnp.int32),
            pltpu.VMEM_SHARED((S, D), jnp.float32),
            pltpu.SemaphoreType.DMA,
            pltpu.SemaphoreType.DMA,
            pltpu.SemaphoreType.DMA,
            pltpu.SemaphoreType.DMA,
        ],
    )(_sc_segsum_body)


_SC_CACHE = {}


def _sc_segsum(*args):
    if "k" not in _SC_CACHE:
        _SC_CACHE["k"] = _make_sc_segsum()
    return _SC_CACHE["k"](*args)


def _counts_body(ids_ref, out_ref, acc_ref):
    i = pl.program_id(0)
    ids_row = ids_ref[0]
    ohT = (ids_row == lax.broadcasted_iota(jnp.int32, (S, R), 0)
           ).astype(jnp.bfloat16)
    part = jnp.dot(ohT, jnp.ones((R, 8), jnp.bfloat16),
                   preferred_element_type=jnp.float32)

    @pl.when(i == 0)
    def _z():
        acc_ref[...] = jnp.zeros_like(acc_ref)

    acc_ref[...] += part

    @pl.when(i == NB - 1)
    def _w():
        out_ref[...] = acc_ref[...]


def _g_body(sums_ref, cnts_ref, w2t_ref, b_ref, gdup_ref, gfull_ref):
    ssum = sums_ref[0] + sums_ref[1]
    counts = cnts_ref[...][:, 0:1]
    mean = ssum * (1.0 / jnp.maximum(counts, 1.0))
    g = jnp.dot(mean, w2t_ref[...],
                preferred_element_type=jnp.float32) + b_ref[...]
    gb = g.astype(jnp.bfloat16)
    gfull_ref[...] = gb
    gdup_ref[0] = gb[0:256]
    gdup_ref[1] = gb[128:384]
    gdup_ref[2] = gb[256:512]
    gdup_ref[3] = jnp.concatenate(
        [gb[384:512], jnp.zeros((128, OUT), jnp.bfloat16)], axis=0)


def _fused_body(pref_ref, x_ref, ids_ref, w1t_ref, gdup_ref, gfull_ref,
                out_ref):
    i = pl.program_id(0)
    q0 = pref_ref[0, i]
    wide = pref_ref[1, i]
    ids_row = ids_ref[0]
    main = jnp.dot(x_ref[...], w1t_ref[...],
                   preferred_element_type=jnp.float32)

    @pl.when(wide == 0)
    def _fast():
        rel = ids_row - q0 * 128
        ohT = (rel == lax.broadcasted_iota(jnp.int32, (GW, R), 0)
               ).astype(jnp.bfloat16)
        gath = lax.dot_general(ohT, gdup_ref[0], (((0,), (0,)), ((), ())),
                               preferred_element_type=jnp.float32)
        out_ref[...] = main + gath

    @pl.when(wide == 1)
    def _slow():
        ohT = (ids_row == lax.broadcasted_iota(jnp.int32, (S, R), 0)
               ).astype(jnp.bfloat16)
        gath = lax.dot_general(ohT, gfull_ref[...], (((0,), (0,)), ((), ())),
                               preferred_element_type=jnp.float32)
        out_ref[...] = main + gath


@jax.jit
def kernel(x, batch_id, W, b):
    ids = batch_id.astype(jnp.int32)
    ids3 = ids.reshape(NB, 1, R)
    wt = W.T
    w1t = wt[:D]
    w2t = wt[D:]
    brow = b.reshape(1, OUT)
    zsum = jnp.zeros((S, D), jnp.float32)

    sums = _sc_segsum(x, ids, zsum)

    cnts = pl.pallas_call(
        _counts_body,
        grid=(NB,),
        in_specs=[pl.BlockSpec((1, 1, R), lambda i: (i, 0, 0))],
        out_specs=pl.BlockSpec((S, 8), lambda i: (0, 0)),
        out_shape=jax.ShapeDtypeStruct((S, 8), jnp.float32),
        scratch_shapes=[pltpu.VMEM((S, 8), jnp.float32)],
    )(ids3)

    gdup, gfull = pl.pallas_call(
        _g_body,
        out_shape=[jax.ShapeDtypeStruct((NWIN, GW, OUT), jnp.bfloat16),
                   jax.ShapeDtypeStruct((S, OUT), jnp.bfloat16)],
    )(sums, cnts, w2t, brow)

    starts = ids[::R]
    ends = ids[R - 1::R]
    q0 = starts // 128
    wide = (ends - q0 * 128 >= GW).astype(jnp.int32)
    pref = jnp.stack([q0, wide])

    grid_spec = pltpu.PrefetchScalarGridSpec(
        num_scalar_prefetch=1,
        grid=(NB,),
        in_specs=[
            pl.BlockSpec((R, D), lambda i, p: (i, 0)),
            pl.BlockSpec((1, 1, R), lambda i, p: (i, 0, 0)),
            pl.BlockSpec((D, OUT), lambda i, p: (0, 0)),
            pl.BlockSpec((1, GW, OUT), lambda i, p: (p[0, i], 0, 0)),
            pl.BlockSpec((S, OUT), lambda i, p: (0, 0)),
        ],
        out_specs=pl.BlockSpec((R, OUT), lambda i, p: (i, 0)),
    )
    return pl.pallas_call(
        _fused_body,
        grid_spec=grid_spec,
        out_shape=jax.ShapeDtypeStruct((N, OUT), jnp.float32),
    )(pref, x, ids3, w1t, gdup, gfull)


# speedup vs baseline: 2.2562x; 1.0179x over previous
import functools

import jax
import jax.numpy as jnp
from jax import lax
from jax.experimental import pallas as pl
from jax.experimental.pallas import tpu as pltpu
from jax.experimental.pallas import tpu_sc as plsc

N = 100000
D = 128
OUT = 128
S = 512
R = 20000
NB = N // R
GW = 256
NWIN = 4

NC = 2
NS = 16
NW = NC * NS
CH = 128
NFULL = N // CH
TAIL = N - NFULL * CH
ITERS = (NFULL + NW - 1) // NW


def _sc_segsum_body(x_hbm, ids_hbm, zsum_hbm, sums_hbm,
                    xbuf, idxbuf, xtail, idxtail, acc,
                    semx0, semx1, semi0, semi1):
    cid = lax.axis_index("c")
    sid = lax.axis_index("s")
    wid = cid * NS + sid
    semx = [semx0, semx1]
    semi = [semi0, semi1]

    @pl.when(sid == 0)
    def _init():
        pltpu.sync_copy(zsum_hbm, acc)

    def _start(slot, c):
        off = c * CH
        pltpu.async_copy(x_hbm.at[pl.ds(off, CH)], xbuf.at[slot], semx[slot])
        pltpu.async_copy(ids_hbm.at[pl.ds(off, CH)], idxbuf.at[slot],
                         semi[slot])

    def _wait(slot, c):
        off = c * CH
        pltpu.make_async_copy(x_hbm.at[pl.ds(off, CH)], xbuf.at[slot],
                              semx[slot]).wait()
        pltpu.make_async_copy(ids_hbm.at[pl.ds(off, CH)], idxbuf.at[slot],
                              semi[slot]).wait()

    for b in range(2):
        cpr = wid + b * NW

        @pl.when(cpr < NFULL)
        def _p(cpr=cpr, b=b):
            _start(b, cpr)

    plsc.subcore_barrier()

    def step(k, carry):
        for b in range(2):
            it = 2 * k + b
            c = wid + it * NW

            @pl.when(c < NFULL)
            def _chunk(c=c, b=b):
                _wait(b, c)
                pltpu.sync_copy(xbuf.at[b], acc.at[idxbuf.at[b]], add=True)
                c2 = c + 2 * NW

                @pl.when(c2 < NFULL)
                def _n(c2=c2, b=b):
                    _start(b, c2)

        return carry

    lax.fori_loop(0, (ITERS + 1) // 2, step, 0)

    @pl.when(wid == 0)
    def _tail():
        pltpu.sync_copy(x_hbm.at[pl.ds(NFULL * CH, TAIL)], xtail)
        pltpu.sync_copy(ids_hbm.at[pl.ds(NFULL * CH, TAIL)], idxtail)
        pltpu.sync_copy(xtail, acc.at[idxtail], add=True)

    plsc.subcore_barrier()

    @pl.when(sid == 0)
    def _out():
        pltpu.sync_copy(acc, sums_hbm.at[cid])


def _make_sc_segsum():
    return functools.partial(
        pl.kernel,
        mesh=plsc.VectorSubcoreMesh(core_axis_name="c", subcore_axis_name="s"),
        out_type=jax.ShapeDtypeStruct((NC, S, D), jnp.float32),
        scratch_types=[
            pltpu.VMEM((2, CH, D), jnp.float32),
            pltpu.VMEM((2, CH), jnp.int32),
            pltpu.VMEM((TAIL, D), jnp.float32),
            pltpu.VMEM((TAIL,), jnp.int32),
            pltpu.VMEM_SHARED((S, D), jnp.float32),
            pltpu.SemaphoreType.DMA,
            pltpu.SemaphoreType.DMA,
            pltpu.SemaphoreType.DMA,
            pltpu.SemaphoreType.DMA,
        ],
    )(_sc_segsum_body)


_SC_CACHE = {}


def _sc_segsum(*args):
    if "k" not in _SC_CACHE:
        _SC_CACHE["k"] = _make_sc_segsum()
    return _SC_CACHE["k"](*args)


def _counts_body(ids_ref, out_ref, acc_ref):
    i = pl.program_id(0)
    ids_row = ids_ref[0]
    ohT = (ids_row == lax.broadcasted_iota(jnp.int32, (S, R), 0)
           ).astype(jnp.bfloat16)
    part = jnp.dot(ohT, jnp.ones((R, 8), jnp.bfloat16),
                   preferred_element_type=jnp.float32)

    @pl.when(i == 0)
    def _z():
        acc_ref[...] = jnp.zeros_like(acc_ref)

    acc_ref[...] += part

    @pl.when(i == NB - 1)
    def _w():
        out_ref[...] = acc_ref[...]


def _fused_body(pref_ref, x_ref, ids_ref, w1t_ref, w2t_ref, b_ref,
                sums_ref, cnts_ref, out_ref, g_ref):
    i = pl.program_id(0)
    q0 = pref_ref[0, i]
    wide = pref_ref[1, i]
    ids_row = ids_ref[0]

    @pl.when(i == 0)
    def _make_g():
        ssum = sums_ref[0] + sums_ref[1]
        counts = cnts_ref[...][:, 0:1]
        mean = ssum * (1.0 / jnp.maximum(counts, 1.0))
        g = jnp.dot(mean, w2t_ref[...],
                    preferred_element_type=jnp.float32) + b_ref[...]
        g_ref[0:S] = g.astype(jnp.bfloat16)
        g_ref[S:] = jnp.zeros((GW - 128, OUT), jnp.bfloat16)

    main = jnp.dot(x_ref[...], w1t_ref[...],
                   preferred_element_type=jnp.float32)

    @pl.when(wide == 0)
    def _fast():
        rel = ids_row - q0 * 128
        ohT = (rel == lax.broadcasted_iota(jnp.int32, (GW, R), 0)
               ).astype(jnp.bfloat16)
        gw = g_ref[pl.ds(q0 * 128, GW)]
        gath = lax.dot_general(ohT, gw, (((0,), (0,)), ((), ())),
                               preferred_element_type=jnp.float32)
        out_ref[...] = main + gath

    @pl.when(wide == 1)
    def _slow():
        ohT = (ids_row == lax.broadcasted_iota(jnp.int32, (S, R), 0)
               ).astype(jnp.bfloat16)
        gath = lax.dot_general(ohT, g_ref[pl.ds(0, S)],
                               (((0,), (0,)), ((), ())),
                               preferred_element_type=jnp.float32)
        out_ref[...] = main + gath


@jax.jit
def kernel(x, batch_id, W, b):
    ids = batch_id.astype(jnp.int32)
    ids3 = ids.reshape(NB, 1, R)
    wt = W.T
    w1t = wt[:D]
    w2t = wt[D:]
    brow = b.reshape(1, OUT)
    zsum = jnp.zeros((S, D), jnp.float32)

    sums = _sc_segsum(x, ids, zsum)

    cnts = pl.pallas_call(
        _counts_body,
        grid=(NB,),
        in_specs=[pl.BlockSpec((1, 1, R), lambda i: (i, 0, 0))],
        out_specs=pl.BlockSpec((S, 8), lambda i: (0, 0)),
        out_shape=jax.ShapeDtypeStruct((S, 8), jnp.float32),
        scratch_shapes=[pltpu.VMEM((S, 8), jnp.float32)],
    )(ids3)

    starts = ids[::R]
    ends = ids[R - 1::R]
    q0 = starts // 128
    wide = (ends - q0 * 128 >= GW).astype(jnp.int32)
    pref = jnp.stack([q0, wide])

    grid_spec = pltpu.PrefetchScalarGridSpec(
        num_scalar_prefetch=1,
        grid=(NB,),
        in_specs=[
            pl.BlockSpec((R, D), lambda i, p: (i, 0)),
            pl.BlockSpec((1, 1, R), lambda i, p: (i, 0, 0)),
            pl.BlockSpec((D, OUT), lambda i, p: (0, 0)),
            pl.BlockSpec((D, OUT), lambda i, p: (0, 0)),
            pl.BlockSpec((1, OUT), lambda i, p: (0, 0)),
            pl.BlockSpec((NC, S, D), lambda i, p: (0, 0, 0)),
            pl.BlockSpec((S, 8), lambda i, p: (0, 0)),
        ],
        out_specs=pl.BlockSpec((R, OUT), lambda i, p: (i, 0)),
        scratch_shapes=[pltpu.VMEM((S + GW - 128, OUT), jnp.bfloat16)],
    )
    return pl.pallas_call(
        _fused_body,
        grid_spec=grid_spec,
        out_shape=jax.ShapeDtypeStruct((N, OUT), jnp.float32),
    )(pref, x, ids3, w1t, w2t, brow, sums, cnts)


# speedup vs baseline: 2.2586x; 1.0010x over previous
import functools

import jax
import jax.numpy as jnp
from jax import lax
from jax.experimental import pallas as pl
from jax.experimental.pallas import tpu as pltpu
from jax.experimental.pallas import tpu_sc as plsc

N = 100000
D = 128
OUT = 128
S = 512
R = 20000
NB = N // R
GW = 256

NC = 2
NS = 16
NW = NC * NS
CH = 128
NFULL = N // CH
TAIL = N - NFULL * CH
ITERS = (NFULL + NW - 1) // NW


def _sc_segsum_body(x_hbm, ids_hbm, zsum_hbm, sums_hbm,
                    xbuf, idxbuf, xtail, idxtail, acc,
                    semx0, semx1, semi0, semi1):
    cid = lax.axis_index("c")
    sid = lax.axis_index("s")
    wid = cid * NS + sid
    semx = [semx0, semx1]
    semi = [semi0, semi1]

    @pl.when(sid == 0)
    def _init():
        pltpu.sync_copy(zsum_hbm, acc)

    def _start(slot, c):
        off = c * CH
        pltpu.async_copy(x_hbm.at[pl.ds(off, CH)], xbuf.at[slot], semx[slot])
        pltpu.async_copy(ids_hbm.at[pl.ds(off, CH)], idxbuf.at[slot],
                         semi[slot])

    def _wait(slot, c):
        off = c * CH
        pltpu.make_async_copy(x_hbm.at[pl.ds(off, CH)], xbuf.at[slot],
                              semx[slot]).wait()
        pltpu.make_async_copy(ids_hbm.at[pl.ds(off, CH)], idxbuf.at[slot],
                              semi[slot]).wait()

    for b in range(2):
        cpr = wid + b * NW

        @pl.when(cpr < NFULL)
        def _p(cpr=cpr, b=b):
            _start(b, cpr)

    plsc.subcore_barrier()

    def step(k, carry):
        for b in range(2):
            it = 2 * k + b
            c = wid + it * NW

            @pl.when(c < NFULL)
            def _chunk(c=c, b=b):
                _wait(b, c)
                pltpu.sync_copy(xbuf.at[b], acc.at[idxbuf.at[b]], add=True)
                c2 = c + 2 * NW

                @pl.when(c2 < NFULL)
                def _n(c2=c2, b=b):
                    _start(b, c2)

        return carry

    lax.fori_loop(0, (ITERS + 1) // 2, step, 0)

    @pl.when(wid == 0)
    def _tail():
        pltpu.sync_copy(x_hbm.at[pl.ds(NFULL * CH, TAIL)], xtail)
        pltpu.sync_copy(ids_hbm.at[pl.ds(NFULL * CH, TAIL)], idxtail)
        pltpu.sync_copy(xtail, acc.at[idxtail], add=True)

    plsc.subcore_barrier()

    @pl.when(sid == 0)
    def _out():
        pltpu.sync_copy(acc, sums_hbm.at[cid])


def _make_sc_segsum():
    return functools.partial(
        pl.kernel,
        mesh=plsc.VectorSubcoreMesh(core_axis_name="c", subcore_axis_name="s"),
        out_type=jax.ShapeDtypeStruct((NC, S, D), jnp.float32),
        scratch_types=[
            pltpu.VMEM((2, CH, D), jnp.float32),
            pltpu.VMEM((2, CH), jnp.int32),
            pltpu.VMEM((TAIL, D), jnp.float32),
            pltpu.VMEM((TAIL,), jnp.int32),
            pltpu.VMEM_SHARED((S, D), jnp.float32),
            pltpu.SemaphoreType.DMA,
            pltpu.SemaphoreType.DMA,
            pltpu.SemaphoreType.DMA,
            pltpu.SemaphoreType.DMA,
        ],
    )(_sc_segsum_body)


_SC_CACHE = {}


def _sc_segsum(*args):
    if "k" not in _SC_CACHE:
        _SC_CACHE["k"] = _make_sc_segsum()
    return _SC_CACHE["k"](*args)


def _counts_body(ids_ref, out_ref, acc_ref):
    i = pl.program_id(0)
    ids_row = ids_ref[0]
    ohT = (ids_row == lax.broadcasted_iota(jnp.int32, (S, R), 0)
           ).astype(jnp.bfloat16)
    part = jnp.dot(ohT, jnp.ones((R, 8), jnp.bfloat16),
                   preferred_element_type=jnp.float32)

    @pl.when(i == 0)
    def _z():
        acc_ref[...] = jnp.zeros_like(acc_ref)

    acc_ref[...] += part

    @pl.when(i == NB - 1)
    def _w():
        out_ref[...] = acc_ref[...]


def _fused_body(pref_ref, x_ref, ids_ref, w1t_ref, w2t_ref, b_ref,
                sums_ref, cnts_ref, out_ref, g_ref):
    i = pl.program_id(0)
    q0 = pref_ref[0, i]
    wide = pref_ref[1, i]
    ids_row = ids_ref[0]

    @pl.when(i == 0)
    def _make_g():
        ssum = sums_ref[0] + sums_ref[1]
        counts = cnts_ref[...][:, 0:1]
        mean = ssum * (1.0 / jnp.maximum(counts, 1.0))
        g = jnp.dot(mean, w2t_ref[...],
                    preferred_element_type=jnp.float32) + b_ref[...]
        g_ref[0:S] = g.astype(jnp.bfloat16)
        g_ref[S:] = jnp.zeros((GW - 128, OUT), jnp.bfloat16)

    main = jnp.dot(x_ref[...], w1t_ref[...],
                   preferred_element_type=jnp.float32)

    @pl.when(wide == 0)
    def _fast():
        rel = ids_row - q0 * 128
        ohT = (rel == lax.broadcasted_iota(jnp.int32, (GW, R), 0)
               ).astype(jnp.bfloat16)
        gw = g_ref[pl.ds(q0 * 128, GW)]
        gath = lax.dot_general(ohT, gw, (((0,), (0,)), ((), ())),
                               preferred_element_type=jnp.float32)
        out_ref[...] = main + gath

    @pl.when(wide == 1)
    def _slow():
        ohT = (ids_row == lax.broadcasted_iota(jnp.int32, (S, R), 0)
               ).astype(jnp.bfloat16)
        gath = lax.dot_general(ohT, g_ref[pl.ds(0, S)],
                               (((0,), (0,)), ((), ())),
                               preferred_element_type=jnp.float32)
        out_ref[...] = main + gath


@jax.jit
def kernel(x, batch_id, W, b):
    ids = batch_id.astype(jnp.int32)
    ids3 = ids.reshape(NB, 1, R)
    wt = W.T
    w1t = wt[:D]
    w2t = wt[D:]
    brow = b.reshape(1, OUT)
    zsum = jnp.zeros((S, D), jnp.float32)

    sums = _sc_segsum(x, ids, zsum)

    cnts = pl.pallas_call(
        _counts_body,
        grid=(NB,),
        in_specs=[pl.BlockSpec((1, 1, R), lambda i: (i, 0, 0))],
        out_specs=pl.BlockSpec((S, 8), lambda i: (0, 0)),
        out_shape=jax.ShapeDtypeStruct((S, 8), jnp.float32),
        scratch_shapes=[pltpu.VMEM((S, 8), jnp.float32)],
    )(ids3)

    starts = ids[::R]
    ends = ids[R - 1::R]
    q0 = starts // 128
    wide = (ends - q0 * 128 >= GW).astype(jnp.int32)
    pref = jnp.stack([q0, wide])

    grid_spec = pltpu.PrefetchScalarGridSpec(
        num_scalar_prefetch=1,
        grid=(NB,),
        in_specs=[
            pl.BlockSpec((R, D), lambda i, p: (i, 0)),
            pl.BlockSpec((1, 1, R), lambda i, p: (i, 0, 0)),
            pl.BlockSpec((D, OUT), lambda i, p: (0, 0)),
            pl.BlockSpec((D, OUT), lambda i, p: (0, 0)),
            pl.BlockSpec((1, OUT), lambda i, p: (0, 0)),
            pl.BlockSpec((NC, S, D), lambda i, p: (0, 0, 0)),
            pl.BlockSpec((S, 8), lambda i, p: (0, 0)),
        ],
        out_specs=pl.BlockSpec((R, OUT), lambda i, p: (i, 0)),
        scratch_shapes=[pltpu.VMEM((S + GW - 128, OUT), jnp.bfloat16)],
    )
    return pl.pallas_call(
        _fused_body,
        grid_spec=grid_spec,
        out_shape=jax.ShapeDtypeStruct((N, OUT), jnp.float32),
    )(pref, x, ids3, w1t, w2t, brow, sums, cnts)
